# baseline (speedup 1.0000x reference)
import jax
import jax.numpy as jnp
from jax.experimental import pallas as pl


def _add_body(x_ref, w_ref, o_ref):
    o_ref[...] = x_ref[...] + w_ref[...]


def kernel(inputs, embed_weight):
    B, T, D = inputs.shape
    BT = 256
    grid = (T // BT,)
    return pl.pallas_call(
        _add_body,
        grid=grid,
        in_specs=[
            pl.BlockSpec((B, BT, D), lambda t: (0, t, 0)),
            pl.BlockSpec((BT, D), lambda t: (t, 0)),
        ],
        out_specs=pl.BlockSpec((B, BT, D), lambda t: (0, t, 0)),
        out_shape=jax.ShapeDtypeStruct((B, T, D), inputs.dtype),
    )(inputs, embed_weight)


# speedup vs baseline: 1.0460x; 1.0460x over previous
import jax
import jax.numpy as jnp
from jax.experimental import pallas as pl


def _add_body(x_ref, w_ref, o_ref):
    o_ref[...] = x_ref[...] + w_ref[...]


def kernel(inputs, embed_weight):
    B, T, D = inputs.shape
    x2 = inputs.reshape(B * T, D)
    out2 = pl.pallas_call(
        _add_body,
        grid=(B,),
        in_specs=[
            pl.BlockSpec((T, D), lambda b: (b, 0)),
            pl.BlockSpec((T, D), lambda b: (0, 0)),
        ],
        out_specs=pl.BlockSpec((T, D), lambda b: (b, 0)),
        out_shape=jax.ShapeDtypeStruct((B * T, D), inputs.dtype),
    )(x2, embed_weight)
    return out2.reshape(B, T, D)
